# Initial kernel scaffold; baseline (speedup 1.0000x reference)
import jax
import jax.numpy as jnp
from jax.experimental import pallas as pl


def kernel(v1, v2, y, idx, memory_v1, memory_v2):
    raise NotImplementedError("write your pallas kernel here")


# speedup vs baseline: 4.6122x; 4.6122x over previous
import functools

import jax
import jax.numpy as jnp
from jax import lax
from jax.experimental import pallas as pl
from jax.experimental.pallas import tpu as pltpu
from jax.experimental.pallas import tpu_sc as plsc

NW = 32
CHUNK = 128
LANES = 16
T = 0.07
MOMENTUM = 0.5


def _iota16():
    return lax.iota(jnp.int32, LANES)


def _splat16(x):
    return jnp.full((LANES,), x, dtype=jnp.int32)


def _dot_chunk(buf, vv, out_v, c_base, scratch_v, col_idx, lane_last, n_groups):
    def g_body(g, carry):
        row0 = g * LANES
        for r in range(LANES):
            ridx = _splat16(row0 + r)
            acc = plsc.load_gather(buf, [ridx, col_idx[0]]) * vv[0]
            for p in range(1, 8):
                acc = acc + plsc.load_gather(buf, [ridx, col_idx[p]]) * vv[p]
            scratch_v[pl.ds(r * LANES, LANES)] = plsc.cumsum(acc)
        res = plsc.load_gather(scratch_v, [lane_last])
        oidx = _splat16(c_base + row0) + _iota16()
        plsc.store_scatter(out_v, [oidx], res)
        return carry

    lax.fori_loop(0, n_groups, g_body, 0)


def _sc_body(v1_hbm, v2_hbm, idx_hbm, y_hbm,
             mem1_hbm, mem2_hbm,
             o1_hbm, o2_hbm, g1_hbm, g2_hbm,
             idx_v, v1_v, v2_v, y_v, gbuf_v,
             b1s0, b1s1, b2s0, b2s1,
             out1_v, out2_v, scratch_v,
             s1s0, s1s1, s2s0, s2s1, gsem):
    B = v1_hbm.shape[0]
    D = v1_hbm.shape[1]
    n_chunks = idx_hbm.shape[1]
    bpw = B // NW
    wid = lax.axis_index("s") * 2 + lax.axis_index("c")
    inv_t = jnp.float32(1.0 / T)

    col_idx = [_iota16() + p * LANES for p in range(8)]
    lane_last = _iota16() * LANES + (LANES - 1)
    n_groups = CHUNK // LANES

    pltpu.sync_copy(y_hbm.at[wid], y_v)
    pltpu.async_copy(mem1_hbm.at[y_v], gbuf_v, gsem).wait()
    pltpu.sync_copy(gbuf_v, g1_hbm.at[pl.ds(wid * bpw, bpw)])
    pltpu.async_copy(mem2_hbm.at[y_v], gbuf_v, gsem).wait()
    pltpu.sync_copy(gbuf_v, g2_hbm.at[pl.ds(wid * bpw, bpw)])

    bufs1 = (b1s0, b1s1)
    bufs2 = (b2s0, b2s1)
    sems1 = (s1s0, s1s1)
    sems2 = (s2s0, s2s1)

    def start(c, slot):
        pltpu.async_copy(mem1_hbm.at[idx_v.at[c]], bufs1[slot], sems1[slot])
        pltpu.async_copy(mem2_hbm.at[idx_v.at[c]], bufs2[slot], sems2[slot])

    def wait(c, slot):
        pltpu.make_async_copy(mem1_hbm.at[idx_v.at[c]], bufs1[slot], sems1[slot]).wait()
        pltpu.make_async_copy(mem2_hbm.at[idx_v.at[c]], bufs2[slot], sems2[slot]).wait()

    def b_body(local, carry):
        b = wid * bpw + local
        pltpu.sync_copy(idx_hbm.at[b], idx_v)
        pltpu.sync_copy(v1_hbm.at[b], v1_v)
        pltpu.sync_copy(v2_hbm.at[b], v2_v)
        vv1 = [v1_v[pl.ds(p * LANES, LANES)] * inv_t for p in range(8)]
        vv2 = [v2_v[pl.ds(p * LANES, LANES)] * inv_t for p in range(8)]

        start(0, 0)

        def c_pair(cp, carry2):
            for par in range(2):
                c = 2 * cp + par

                @pl.when(c + 1 < n_chunks)
                def _():
                    start(c + 1, 1 - par)

                wait(c, par)
                _dot_chunk(bufs1[par], vv2, out2_v, c * CHUNK, scratch_v,
                           col_idx, lane_last, n_groups)
                _dot_chunk(bufs2[par], vv1, out1_v, c * CHUNK, scratch_v,
                           col_idx, lane_last, n_groups)
            return carry2

        lax.fori_loop(0, n_chunks // 2, c_pair, 0)
        pltpu.sync_copy(out1_v, o1_hbm.at[b])
        pltpu.sync_copy(out2_v, o2_hbm.at[b])
        return carry

    lax.fori_loop(0, bpw, b_body, 0)


def _make_sc_call(B, K1, D, N, interpret=False):
    n_chunks = K1 // CHUNK
    bpw = B // NW
    mesh = plsc.VectorSubcoreMesh(core_axis_name="c", subcore_axis_name="s",
                                  num_cores=2, num_subcores=16)
    return pl.kernel(
        _sc_body,
        out_type=(
            jax.ShapeDtypeStruct((B, K1), jnp.float32),
            jax.ShapeDtypeStruct((B, K1), jnp.float32),
            jax.ShapeDtypeStruct((B, D), jnp.float32),
            jax.ShapeDtypeStruct((B, D), jnp.float32),
        ),
        mesh=mesh,
        scratch_types=[
            pltpu.VMEM((n_chunks, CHUNK), jnp.int32),
            pltpu.VMEM((D,), jnp.float32),
            pltpu.VMEM((D,), jnp.float32),
            pltpu.VMEM((bpw,), jnp.int32),
            pltpu.VMEM((bpw, D), jnp.float32),
            pltpu.VMEM((CHUNK, D), jnp.float32),
            pltpu.VMEM((CHUNK, D), jnp.float32),
            pltpu.VMEM((CHUNK, D), jnp.float32),
            pltpu.VMEM((CHUNK, D), jnp.float32),
            pltpu.VMEM((K1,), jnp.float32),
            pltpu.VMEM((K1,), jnp.float32),
            pltpu.VMEM((LANES * LANES,), jnp.float32),
            pltpu.SemaphoreType.DMA,
            pltpu.SemaphoreType.DMA,
            pltpu.SemaphoreType.DMA,
            pltpu.SemaphoreType.DMA,
            pltpu.SemaphoreType.DMA,
        ],
        compiler_params=pltpu.CompilerParams(needs_layout_passes=False),
        interpret=interpret,
    )


LAG = 32


def _tc_update_body(y_ref, w_ref, g1_ref, g2_ref, v1_ref, v2_ref,
                    m1_ref, m2_ref, o1_ref, o2_ref,
                    u1_ref, u2_ref, sem1, sem2):
    del m1_ref, m2_ref
    B = y_ref.shape[0]
    t1 = g1_ref[...] * MOMENTUM + v1_ref[...] * (1.0 - MOMENTUM)
    n1 = jnp.sum(t1 * t1, axis=1, keepdims=True)
    u1_ref[...] = t1 / jnp.sqrt(n1)
    t2 = g2_ref[...] * MOMENTUM + v2_ref[...] * (1.0 - MOMENTUM)
    n2 = jnp.sum(t2 * t2, axis=1, keepdims=True)
    u2_ref[...] = t2 / jnp.sqrt(n2)

    def _wait_one():
        pltpu.make_async_copy(u1_ref.at[pl.ds(0, 1)], o1_ref.at[pl.ds(0, 1)], sem1).wait()
        pltpu.make_async_copy(u2_ref.at[pl.ds(0, 1)], o2_ref.at[pl.ds(0, 1)], sem2).wait()

    def sbody(i, carry):
        yi = y_ref[i]
        wi = w_ref[i]
        pltpu.make_async_copy(u1_ref.at[pl.ds(wi, 1)], o1_ref.at[pl.ds(yi, 1)], sem1).start()
        pltpu.make_async_copy(u2_ref.at[pl.ds(wi, 1)], o2_ref.at[pl.ds(yi, 1)], sem2).start()

        @pl.when(i >= LAG)
        def _():
            _wait_one()

        return carry

    lax.fori_loop(0, B, sbody, 0)

    def dbody(i, carry):
        _wait_one()
        return carry

    lax.fori_loop(0, min(LAG, B), dbody, 0)


def _make_tc_update(B, D, N, interpret=False):
    return pl.pallas_call(
        _tc_update_body,
        out_shape=(
            jax.ShapeDtypeStruct((N, D), jnp.float32),
            jax.ShapeDtypeStruct((N, D), jnp.float32),
        ),
        in_specs=[
            pl.BlockSpec(memory_space=pltpu.MemorySpace.SMEM),
            pl.BlockSpec(memory_space=pltpu.MemorySpace.SMEM),
            pl.BlockSpec(memory_space=pltpu.MemorySpace.VMEM),
            pl.BlockSpec(memory_space=pltpu.MemorySpace.VMEM),
            pl.BlockSpec(memory_space=pltpu.MemorySpace.VMEM),
            pl.BlockSpec(memory_space=pltpu.MemorySpace.VMEM),
            pl.BlockSpec(memory_space=pltpu.MemorySpace.HBM),
            pl.BlockSpec(memory_space=pltpu.MemorySpace.HBM),
        ],
        out_specs=(
            pl.BlockSpec(memory_space=pltpu.MemorySpace.HBM),
            pl.BlockSpec(memory_space=pltpu.MemorySpace.HBM),
        ),
        scratch_shapes=[
            pltpu.VMEM((B, D), jnp.float32),
            pltpu.VMEM((B, D), jnp.float32),
            pltpu.SemaphoreType.DMA,
            pltpu.SemaphoreType.DMA,
        ],
        input_output_aliases={6: 0, 7: 1},
        interpret=interpret,
    )


def _impl(v1, v2, y, idx, memory_v1, memory_v2, interpret=False):
    B, D = v1.shape
    K1 = idx.shape[1]
    N = memory_v1.shape[0]

    iota_b = jnp.arange(B, dtype=jnp.int32)
    w = jnp.zeros((N,), jnp.int32).at[y].max(iota_b)[y]

    sc_call = _make_sc_call(B, K1, D, N, interpret=interpret)
    out1, out2, g1, g2 = sc_call(
        v1, v2,
        idx.reshape(B, K1 // CHUNK, CHUNK),
        y.reshape(NW, B // NW),
        memory_v1, memory_v2,
    )

    tc_call = _make_tc_update(B, D, N, interpret=interpret)
    new1, new2 = tc_call(y, w, g1, g2, v1, v2, memory_v1, memory_v2)

    return (out1[:, :, None], out2[:, :, None], new1, new2)


def kernel(v1, v2, y, idx, memory_v1, memory_v2):
    return _impl(v1, v2, y, idx, memory_v1, memory_v2, interpret=False)


# speedup vs baseline: 7.3468x; 1.5929x over previous
import functools

import jax
import jax.numpy as jnp
from jax import lax
from jax.experimental import pallas as pl
from jax.experimental.pallas import tpu as pltpu
from jax.experimental.pallas import tpu_sc as plsc

NW = 32
CHUNK = 128
LANES = 16
T = 0.07
MOMENTUM = 0.5


def _iota16():
    return lax.iota(jnp.int32, LANES)


def _splat16(x):
    return jnp.full((LANES,), x, dtype=jnp.int32)


def _dot_chunk(buf, vv, out_v, c_base, scratch_v, col_idx, lane_base, n_groups):
    def g_body(g, carry):
        row0 = g * LANES
        for r in range(0, LANES, 2):
            ra = row0 + r
            rb = ra + 1
            la = [buf[ra, pl.ds(p * LANES, LANES)] for p in range(8)]
            lb = [buf[rb, pl.ds(p * LANES, LANES)] for p in range(8)]
            a = [la[p] * vv[p] for p in range(4)]
            b = [lb[p] * vv[p] for p in range(4)]
            for p in range(4, 8):
                a[p - 4] = a[p - 4] + la[p] * vv[p]
                b[p - 4] = b[p - 4] + lb[p] * vv[p]
            scratch_v[pl.ds(r * LANES, LANES)] = (a[0] + a[1]) + (a[2] + a[3])
            scratch_v[pl.ds((r + 1) * LANES, LANES)] = (b[0] + b[1]) + (b[2] + b[3])
        tot = plsc.load_gather(scratch_v, [lane_base])
        for c in range(1, LANES):
            tot = tot + plsc.load_gather(scratch_v, [lane_base + c])
        out_v[pl.ds(c_base + row0, LANES)] = tot
        return carry

    lax.fori_loop(0, n_groups, g_body, 0)


def _sc_body(v1_hbm, v2_hbm, idx_hbm, y_hbm,
             mem1_hbm, mem2_hbm,
             o1_hbm, o2_hbm, g1_hbm, g2_hbm,
             idx_v, v1_v, v2_v, y_v, gbuf_v,
             b1s0, b1s1, b2s0, b2s1,
             out1_v, out2_v, scratch_v,
             s1s0, s1s1, s2s0, s2s1, gsem):
    B = v1_hbm.shape[0]
    D = v1_hbm.shape[1]
    n_chunks = idx_hbm.shape[1]
    bpw = B // NW
    wid = lax.axis_index("s") * 2 + lax.axis_index("c")
    inv_t = jnp.float32(1.0 / T)

    col_idx = [_iota16() + p * LANES for p in range(8)]
    lane_base = _iota16() * LANES
    n_groups = CHUNK // LANES

    pltpu.sync_copy(y_hbm.at[wid], y_v)
    pltpu.async_copy(mem1_hbm.at[y_v], gbuf_v, gsem).wait()
    pltpu.sync_copy(gbuf_v, g1_hbm.at[pl.ds(wid * bpw, bpw)])
    pltpu.async_copy(mem2_hbm.at[y_v], gbuf_v, gsem).wait()
    pltpu.sync_copy(gbuf_v, g2_hbm.at[pl.ds(wid * bpw, bpw)])

    bufs1 = (b1s0, b1s1)
    bufs2 = (b2s0, b2s1)
    sems1 = (s1s0, s1s1)
    sems2 = (s2s0, s2s1)

    def start(c, slot):
        pltpu.async_copy(mem1_hbm.at[idx_v.at[c]], bufs1[slot], sems1[slot])
        pltpu.async_copy(mem2_hbm.at[idx_v.at[c]], bufs2[slot], sems2[slot])

    def wait(c, slot):
        pltpu.make_async_copy(mem1_hbm.at[idx_v.at[c]], bufs1[slot], sems1[slot]).wait()
        pltpu.make_async_copy(mem2_hbm.at[idx_v.at[c]], bufs2[slot], sems2[slot]).wait()

    def b_body(local, carry):
        b = wid * bpw + local
        pltpu.sync_copy(idx_hbm.at[b], idx_v)
        pltpu.sync_copy(v1_hbm.at[b], v1_v)
        pltpu.sync_copy(v2_hbm.at[b], v2_v)
        vv1 = [v1_v[pl.ds(p * LANES, LANES)] * inv_t for p in range(8)]
        vv2 = [v2_v[pl.ds(p * LANES, LANES)] * inv_t for p in range(8)]

        start(0, 0)

        def c_pair(cp, carry2):
            for par in range(2):
                c = 2 * cp + par

                @pl.when(c + 1 < n_chunks)
                def _():
                    start(c + 1, 1 - par)

                wait(c, par)
                _dot_chunk(bufs1[par], vv2, out2_v, c * CHUNK, scratch_v,
                           col_idx, lane_base, n_groups)
                _dot_chunk(bufs2[par], vv1, out1_v, c * CHUNK, scratch_v,
                           col_idx, lane_base, n_groups)
            return carry2

        lax.fori_loop(0, n_chunks // 2, c_pair, 0)
        pltpu.sync_copy(out1_v, o1_hbm.at[b])
        pltpu.sync_copy(out2_v, o2_hbm.at[b])
        return carry

    lax.fori_loop(0, bpw, b_body, 0)


def _make_sc_call(B, K1, D, N, interpret=False):
    n_chunks = K1 // CHUNK
    bpw = B // NW
    mesh = plsc.VectorSubcoreMesh(core_axis_name="c", subcore_axis_name="s",
                                  num_cores=2, num_subcores=16)
    return pl.kernel(
        _sc_body,
        out_type=(
            jax.ShapeDtypeStruct((B, K1), jnp.float32),
            jax.ShapeDtypeStruct((B, K1), jnp.float32),
            jax.ShapeDtypeStruct((B, D), jnp.float32),
            jax.ShapeDtypeStruct((B, D), jnp.float32),
        ),
        mesh=mesh,
        scratch_types=[
            pltpu.VMEM((n_chunks, CHUNK), jnp.int32),
            pltpu.VMEM((D,), jnp.float32),
            pltpu.VMEM((D,), jnp.float32),
            pltpu.VMEM((bpw,), jnp.int32),
            pltpu.VMEM((bpw, D), jnp.float32),
            pltpu.VMEM((CHUNK, D), jnp.float32),
            pltpu.VMEM((CHUNK, D), jnp.float32),
            pltpu.VMEM((CHUNK, D), jnp.float32),
            pltpu.VMEM((CHUNK, D), jnp.float32),
            pltpu.VMEM((K1,), jnp.float32),
            pltpu.VMEM((K1,), jnp.float32),
            pltpu.VMEM((LANES * LANES,), jnp.float32),
            pltpu.SemaphoreType.DMA,
            pltpu.SemaphoreType.DMA,
            pltpu.SemaphoreType.DMA,
            pltpu.SemaphoreType.DMA,
            pltpu.SemaphoreType.DMA,
        ],
        compiler_params=pltpu.CompilerParams(needs_layout_passes=False),
        interpret=interpret,
    )


LAG = 32


def _tc_update_body(y_ref, w_ref, g1_ref, g2_ref, v1_ref, v2_ref,
                    m1_ref, m2_ref, o1_ref, o2_ref,
                    u1_ref, u2_ref, sem1, sem2):
    del m1_ref, m2_ref
    B = y_ref.shape[0]
    t1 = g1_ref[...] * MOMENTUM + v1_ref[...] * (1.0 - MOMENTUM)
    n1 = jnp.sum(t1 * t1, axis=1, keepdims=True)
    u1_ref[...] = t1 / jnp.sqrt(n1)
    t2 = g2_ref[...] * MOMENTUM + v2_ref[...] * (1.0 - MOMENTUM)
    n2 = jnp.sum(t2 * t2, axis=1, keepdims=True)
    u2_ref[...] = t2 / jnp.sqrt(n2)

    def _wait_one():
        pltpu.make_async_copy(u1_ref.at[pl.ds(0, 1)], o1_ref.at[pl.ds(0, 1)], sem1).wait()
        pltpu.make_async_copy(u2_ref.at[pl.ds(0, 1)], o2_ref.at[pl.ds(0, 1)], sem2).wait()

    def sbody(i, carry):
        yi = y_ref[i]
        wi = w_ref[i]
        pltpu.make_async_copy(u1_ref.at[pl.ds(wi, 1)], o1_ref.at[pl.ds(yi, 1)], sem1).start()
        pltpu.make_async_copy(u2_ref.at[pl.ds(wi, 1)], o2_ref.at[pl.ds(yi, 1)], sem2).start()

        @pl.when(i >= LAG)
        def _():
            _wait_one()

        return carry

    lax.fori_loop(0, B, sbody, 0)

    def dbody(i, carry):
        _wait_one()
        return carry

    lax.fori_loop(0, min(LAG, B), dbody, 0)


def _make_tc_update(B, D, N, interpret=False):
    return pl.pallas_call(
        _tc_update_body,
        out_shape=(
            jax.ShapeDtypeStruct((N, D), jnp.float32),
            jax.ShapeDtypeStruct((N, D), jnp.float32),
        ),
        in_specs=[
            pl.BlockSpec(memory_space=pltpu.MemorySpace.SMEM),
            pl.BlockSpec(memory_space=pltpu.MemorySpace.SMEM),
            pl.BlockSpec(memory_space=pltpu.MemorySpace.VMEM),
            pl.BlockSpec(memory_space=pltpu.MemorySpace.VMEM),
            pl.BlockSpec(memory_space=pltpu.MemorySpace.VMEM),
            pl.BlockSpec(memory_space=pltpu.MemorySpace.VMEM),
            pl.BlockSpec(memory_space=pltpu.MemorySpace.HBM),
            pl.BlockSpec(memory_space=pltpu.MemorySpace.HBM),
        ],
        out_specs=(
            pl.BlockSpec(memory_space=pltpu.MemorySpace.HBM),
            pl.BlockSpec(memory_space=pltpu.MemorySpace.HBM),
        ),
        scratch_shapes=[
            pltpu.VMEM((B, D), jnp.float32),
            pltpu.VMEM((B, D), jnp.float32),
            pltpu.SemaphoreType.DMA,
            pltpu.SemaphoreType.DMA,
        ],
        input_output_aliases={6: 0, 7: 1},
        interpret=interpret,
    )


def _impl(v1, v2, y, idx, memory_v1, memory_v2, interpret=False):
    B, D = v1.shape
    K1 = idx.shape[1]
    N = memory_v1.shape[0]

    iota_b = jnp.arange(B, dtype=jnp.int32)
    w = jnp.zeros((N,), jnp.int32).at[y].max(iota_b)[y]

    sc_call = _make_sc_call(B, K1, D, N, interpret=interpret)
    out1, out2, g1, g2 = sc_call(
        v1, v2,
        idx.reshape(B, K1 // CHUNK, CHUNK),
        y.reshape(NW, B // NW),
        memory_v1, memory_v2,
    )

    tc_call = _make_tc_update(B, D, N, interpret=interpret)
    new1, new2 = tc_call(y, w, g1, g2, v1, v2, memory_v1, memory_v2)

    return (out1[:, :, None], out2[:, :, None], new1, new2)


def kernel(v1, v2, y, idx, memory_v1, memory_v2):
    return _impl(v1, v2, y, idx, memory_v1, memory_v2, interpret=False)


# speedup vs baseline: 8.2835x; 1.1275x over previous
import functools

import jax
import jax.numpy as jnp
from jax import lax
from jax.experimental import pallas as pl
from jax.experimental.pallas import tpu as pltpu
from jax.experimental.pallas import tpu_sc as plsc

NW = 32
CHUNK = 128
LANES = 16
T = 0.07
MOMENTUM = 0.5


def _iota16():
    return lax.iota(jnp.int32, LANES)


def _splat16(x):
    return jnp.full((LANES,), x, dtype=jnp.int32)


def _dot_chunk(buf, vv, out_v, c_base, scratch_v, col_idx, lane_base, n_groups):
    ones = jnp.ones((LANES,), dtype=jnp.int32)

    def g_body(g, carry):
        row0 = g * LANES
        for r in range(0, LANES, 4):
            rows = [row0 + r + j for j in range(4)]
            ld = [[buf[rw, pl.ds(p * LANES, LANES)] for p in range(8)]
                  for rw in rows]
            for j in range(4):
                a = [ld[j][p] * vv[p] for p in range(4)]
                for p in range(4, 8):
                    a[p - 4] = a[p - 4] + ld[j][p] * vv[p]
                scratch_v[pl.ds((r + j) * LANES, LANES)] = (a[0] + a[1]) + (a[2] + a[3])
        idxc = lane_base
        tot = plsc.load_gather(scratch_v, [idxc])
        for c in range(1, LANES):
            idxc = idxc + ones
            tot = tot + plsc.load_gather(scratch_v, [idxc])
        out_v[pl.ds(c_base + row0, LANES)] = tot
        return carry

    lax.fori_loop(0, n_groups, g_body, 0)


def _sc_body(v1_hbm, v2_hbm, idx_hbm, y_hbm,
             mem1_hbm, mem2_hbm,
             o1_hbm, o2_hbm, g1_hbm, g2_hbm,
             idx_v, v1_v, v2_v, y_v, gbuf_v,
             b1s0, b1s1, b2s0, b2s1,
             out1_v, out2_v, scratch_v,
             s1s0, s1s1, s2s0, s2s1, gsem):
    B = v1_hbm.shape[0]
    D = v1_hbm.shape[1]
    n_chunks = idx_hbm.shape[1]
    bpw = B // NW
    wid = lax.axis_index("s") * 2 + lax.axis_index("c")
    inv_t = jnp.float32(1.0 / T)

    col_idx = [_iota16() + p * LANES for p in range(8)]
    lane_base = _iota16() * LANES
    n_groups = CHUNK // LANES

    pltpu.sync_copy(y_hbm.at[wid], y_v)
    pltpu.async_copy(mem1_hbm.at[y_v], gbuf_v, gsem).wait()
    pltpu.sync_copy(gbuf_v, g1_hbm.at[pl.ds(wid * bpw, bpw)])
    pltpu.async_copy(mem2_hbm.at[y_v], gbuf_v, gsem).wait()
    pltpu.sync_copy(gbuf_v, g2_hbm.at[pl.ds(wid * bpw, bpw)])

    bufs1 = (b1s0, b1s1)
    bufs2 = (b2s0, b2s1)
    sems1 = (s1s0, s1s1)
    sems2 = (s2s0, s2s1)

    def start(c, slot):
        pltpu.async_copy(mem1_hbm.at[idx_v.at[c]], bufs1[slot], sems1[slot])
        pltpu.async_copy(mem2_hbm.at[idx_v.at[c]], bufs2[slot], sems2[slot])

    def wait(c, slot):
        pltpu.make_async_copy(mem1_hbm.at[idx_v.at[c]], bufs1[slot], sems1[slot]).wait()
        pltpu.make_async_copy(mem2_hbm.at[idx_v.at[c]], bufs2[slot], sems2[slot]).wait()

    def b_body(local, carry):
        b = wid * bpw + local
        pltpu.sync_copy(idx_hbm.at[b], idx_v)
        pltpu.sync_copy(v1_hbm.at[b], v1_v)
        pltpu.sync_copy(v2_hbm.at[b], v2_v)
        vv1 = [v1_v[pl.ds(p * LANES, LANES)] * inv_t for p in range(8)]
        vv2 = [v2_v[pl.ds(p * LANES, LANES)] * inv_t for p in range(8)]

        start(0, 0)

        def c_pair(cp, carry2):
            for par in range(2):
                c = 2 * cp + par

                @pl.when(c + 1 < n_chunks)
                def _():
                    start(c + 1, 1 - par)

                wait(c, par)
                _dot_chunk(bufs1[par], vv2, out2_v, c * CHUNK, scratch_v,
                           col_idx, lane_base, n_groups)
                _dot_chunk(bufs2[par], vv1, out1_v, c * CHUNK, scratch_v,
                           col_idx, lane_base, n_groups)
            return carry2

        lax.fori_loop(0, n_chunks // 2, c_pair, 0)
        pltpu.sync_copy(out1_v, o1_hbm.at[b])
        pltpu.sync_copy(out2_v, o2_hbm.at[b])
        return carry

    lax.fori_loop(0, bpw, b_body, 0)


def _make_sc_call(B, K1, D, N, interpret=False):
    n_chunks = K1 // CHUNK
    bpw = B // NW
    mesh = plsc.VectorSubcoreMesh(core_axis_name="c", subcore_axis_name="s",
                                  num_cores=2, num_subcores=16)
    return pl.kernel(
        _sc_body,
        out_type=(
            jax.ShapeDtypeStruct((B, K1), jnp.float32),
            jax.ShapeDtypeStruct((B, K1), jnp.float32),
            jax.ShapeDtypeStruct((B, D), jnp.float32),
            jax.ShapeDtypeStruct((B, D), jnp.float32),
        ),
        mesh=mesh,
        scratch_types=[
            pltpu.VMEM((n_chunks, CHUNK), jnp.int32),
            pltpu.VMEM((D,), jnp.float32),
            pltpu.VMEM((D,), jnp.float32),
            pltpu.VMEM((bpw,), jnp.int32),
            pltpu.VMEM((bpw, D), jnp.float32),
            pltpu.VMEM((CHUNK, D), jnp.float32),
            pltpu.VMEM((CHUNK, D), jnp.float32),
            pltpu.VMEM((CHUNK, D), jnp.float32),
            pltpu.VMEM((CHUNK, D), jnp.float32),
            pltpu.VMEM((K1,), jnp.float32),
            pltpu.VMEM((K1,), jnp.float32),
            pltpu.VMEM((LANES * LANES,), jnp.float32),
            pltpu.SemaphoreType.DMA,
            pltpu.SemaphoreType.DMA,
            pltpu.SemaphoreType.DMA,
            pltpu.SemaphoreType.DMA,
            pltpu.SemaphoreType.DMA,
        ],
        compiler_params=pltpu.CompilerParams(needs_layout_passes=False),
        interpret=interpret,
    )


LAG = 32


def _tc_update_body(y_ref, w_ref, g1_ref, g2_ref, v1_ref, v2_ref,
                    m1_ref, m2_ref, o1_ref, o2_ref,
                    u1_ref, u2_ref, sem1, sem2):
    del m1_ref, m2_ref
    B = y_ref.shape[0]
    t1 = g1_ref[...] * MOMENTUM + v1_ref[...] * (1.0 - MOMENTUM)
    n1 = jnp.sum(t1 * t1, axis=1, keepdims=True)
    u1_ref[...] = t1 / jnp.sqrt(n1)
    t2 = g2_ref[...] * MOMENTUM + v2_ref[...] * (1.0 - MOMENTUM)
    n2 = jnp.sum(t2 * t2, axis=1, keepdims=True)
    u2_ref[...] = t2 / jnp.sqrt(n2)

    def _wait_one():
        pltpu.make_async_copy(u1_ref.at[pl.ds(0, 1)], o1_ref.at[pl.ds(0, 1)], sem1).wait()
        pltpu.make_async_copy(u2_ref.at[pl.ds(0, 1)], o2_ref.at[pl.ds(0, 1)], sem2).wait()

    def sbody(i, carry):
        yi = y_ref[i]
        wi = w_ref[i]
        pltpu.make_async_copy(u1_ref.at[pl.ds(wi, 1)], o1_ref.at[pl.ds(yi, 1)], sem1).start()
        pltpu.make_async_copy(u2_ref.at[pl.ds(wi, 1)], o2_ref.at[pl.ds(yi, 1)], sem2).start()

        @pl.when(i >= LAG)
        def _():
            _wait_one()

        return carry

    lax.fori_loop(0, B, sbody, 0)

    def dbody(i, carry):
        _wait_one()
        return carry

    lax.fori_loop(0, min(LAG, B), dbody, 0)


def _make_tc_update(B, D, N, interpret=False):
    return pl.pallas_call(
        _tc_update_body,
        out_shape=(
            jax.ShapeDtypeStruct((N, D), jnp.float32),
            jax.ShapeDtypeStruct((N, D), jnp.float32),
        ),
        in_specs=[
            pl.BlockSpec(memory_space=pltpu.MemorySpace.SMEM),
            pl.BlockSpec(memory_space=pltpu.MemorySpace.SMEM),
            pl.BlockSpec(memory_space=pltpu.MemorySpace.VMEM),
            pl.BlockSpec(memory_space=pltpu.MemorySpace.VMEM),
            pl.BlockSpec(memory_space=pltpu.MemorySpace.VMEM),
            pl.BlockSpec(memory_space=pltpu.MemorySpace.VMEM),
            pl.BlockSpec(memory_space=pltpu.MemorySpace.HBM),
            pl.BlockSpec(memory_space=pltpu.MemorySpace.HBM),
        ],
        out_specs=(
            pl.BlockSpec(memory_space=pltpu.MemorySpace.HBM),
            pl.BlockSpec(memory_space=pltpu.MemorySpace.HBM),
        ),
        scratch_shapes=[
            pltpu.VMEM((B, D), jnp.float32),
            pltpu.VMEM((B, D), jnp.float32),
            pltpu.SemaphoreType.DMA,
            pltpu.SemaphoreType.DMA,
        ],
        input_output_aliases={6: 0, 7: 1},
        interpret=interpret,
    )


def _impl(v1, v2, y, idx, memory_v1, memory_v2, interpret=False):
    B, D = v1.shape
    K1 = idx.shape[1]
    N = memory_v1.shape[0]

    iota_b = jnp.arange(B, dtype=jnp.int32)
    w = jnp.zeros((N,), jnp.int32).at[y].max(iota_b)[y]

    sc_call = _make_sc_call(B, K1, D, N, interpret=interpret)
    out1, out2, g1, g2 = sc_call(
        v1, v2,
        idx.reshape(B, K1 // CHUNK, CHUNK),
        y.reshape(NW, B // NW),
        memory_v1, memory_v2,
    )

    tc_call = _make_tc_update(B, D, N, interpret=interpret)
    new1, new2 = tc_call(y, w, g1, g2, v1, v2, memory_v1, memory_v2)

    return (out1[:, :, None], out2[:, :, None], new1, new2)


def kernel(v1, v2, y, idx, memory_v1, memory_v2):
    return _impl(v1, v2, y, idx, memory_v1, memory_v2, interpret=False)


# speedup vs baseline: 9.9862x; 1.2056x over previous
import functools

import jax
import jax.numpy as jnp
from jax import lax
from jax.experimental import pallas as pl
from jax.experimental.pallas import tpu as pltpu
from jax.experimental.pallas import tpu_sc as plsc

NW = 32
CHUNK = 128
LANES = 16
T = 0.07
MOMENTUM = 0.5


def _iota16():
    return lax.iota(jnp.int32, LANES)


def _splat16(x):
    return jnp.full((LANES,), x, dtype=jnp.int32)


def _dot_chunk(buf, vv, out_v, c_base, scratch_v, col_idx, lane_base, n_groups):
    ones = jnp.ones((LANES,), dtype=jnp.int32)

    def g_body(g, carry):
        row0 = g * LANES
        for r in range(0, LANES, 4):
            rows = [row0 + r + j for j in range(4)]
            ld = [[buf[rw, pl.ds(p * LANES, LANES)] for p in range(8)]
                  for rw in rows]
            for j in range(4):
                a = [ld[j][p] * vv[p] for p in range(4)]
                for p in range(4, 8):
                    a[p - 4] = a[p - 4] + ld[j][p] * vv[p]
                scratch_v[pl.ds((r + j) * LANES, LANES)] = (a[0] + a[1]) + (a[2] + a[3])
        idxc = lane_base
        tot = plsc.load_gather(scratch_v, [idxc])
        for c in range(1, LANES):
            idxc = idxc + ones
            tot = tot + plsc.load_gather(scratch_v, [idxc])
        out_v[pl.ds(c_base + row0, LANES)] = tot
        return carry

    lax.fori_loop(0, n_groups, g_body, 0)


def _sc_body(v1_hbm, v2_hbm, idx_hbm, y_hbm,
             mem1_hbm, mem2_hbm,
             o1_hbm, o2_hbm, g1_hbm, g2_hbm,
             idx_all, vb1, vb2, y_v, gbuf_v,
             b1s0, b1s1, b2s0, b2s1,
             ob1s0, ob1s1, ob2s0, ob2s1, scratch_v,
             s1s0, s1s1, s2s0, s2s1,
             os1s0, os1s1, os2s0, os2s1, gsem):
    NWw, bpw, n_chunks, _ = idx_hbm.shape
    D = vb1.shape[1]
    wid = lax.axis_index("s") * 2 + lax.axis_index("c")
    inv_t = jnp.float32(1.0 / T)
    cshift = n_chunks.bit_length() - 1
    cmask = n_chunks - 1

    col_idx = [_iota16() + p * LANES for p in range(8)]
    lane_base = _iota16() * LANES
    n_groups = CHUNK // LANES

    pltpu.sync_copy(idx_hbm.at[wid], idx_all)
    pltpu.sync_copy(v1_hbm.at[wid], vb1)
    pltpu.sync_copy(v2_hbm.at[wid], vb2)

    pltpu.sync_copy(y_hbm.at[wid], y_v)
    pltpu.async_copy(mem1_hbm.at[y_v], gbuf_v, gsem).wait()
    pltpu.sync_copy(gbuf_v, g1_hbm.at[pl.ds(wid * bpw, bpw)])
    pltpu.async_copy(mem2_hbm.at[y_v], gbuf_v, gsem).wait()
    pltpu.sync_copy(gbuf_v, g2_hbm.at[pl.ds(wid * bpw, bpw)])

    bufs1 = (b1s0, b1s1)
    bufs2 = (b2s0, b2s1)
    sems1 = (s1s0, s1s1)
    sems2 = (s2s0, s2s1)
    obufs1 = (ob1s0, ob1s1)
    obufs2 = (ob2s0, ob2s1)
    osems1 = (os1s0, os1s1)
    osems2 = (os2s0, os2s1)

    n_units = bpw * n_chunks

    def unit_bc(u):
        return lax.shift_right_logical(u, cshift), jnp.bitwise_and(u, cmask)

    def start(u, slot):
        bl, c = unit_bc(u)
        pltpu.async_copy(mem1_hbm.at[idx_all.at[bl, c]], bufs1[slot], sems1[slot])
        pltpu.async_copy(mem2_hbm.at[idx_all.at[bl, c]], bufs2[slot], sems2[slot])

    def wait(u, slot):
        bl, c = unit_bc(u)
        pltpu.make_async_copy(mem1_hbm.at[idx_all.at[bl, c]], bufs1[slot], sems1[slot]).wait()
        pltpu.make_async_copy(mem2_hbm.at[idx_all.at[bl, c]], bufs2[slot], sems2[slot]).wait()

    def owait(par):
        pltpu.make_async_copy(obufs1[par], o1_hbm.at[0, pl.ds(0, CHUNK)], osems1[par]).wait()
        pltpu.make_async_copy(obufs2[par], o2_hbm.at[0, pl.ds(0, CHUNK)], osems2[par]).wait()

    def u_pair(up, carry2):
        for par in range(2):
            u = 2 * up + par

            @pl.when(u + 1 < n_units)
            def _():
                start(u + 1, 1 - par)

            wait(u, par)

            @pl.when(u >= 2)
            def _():
                owait(par)

            bl, c = unit_bc(u)
            vv1 = [vb1[bl, pl.ds(p * LANES, LANES)] * inv_t for p in range(8)]
            vv2 = [vb2[bl, pl.ds(p * LANES, LANES)] * inv_t for p in range(8)]
            _dot_chunk(bufs1[par], vv2, obufs2[par], 0, scratch_v,
                       col_idx, lane_base, n_groups)
            _dot_chunk(bufs2[par], vv1, obufs1[par], 0, scratch_v,
                       col_idx, lane_base, n_groups)
            b = wid * bpw + bl
            pltpu.async_copy(obufs1[par], o1_hbm.at[b, pl.ds(c * CHUNK, CHUNK)], osems1[par])
            pltpu.async_copy(obufs2[par], o2_hbm.at[b, pl.ds(c * CHUNK, CHUNK)], osems2[par])
        return carry2

    start(0, 0)
    lax.fori_loop(0, n_units // 2, u_pair, 0)
    owait(0)
    owait(1)


def _make_sc_call(B, K1, D, N, interpret=False):
    n_chunks = K1 // CHUNK
    bpw = B // NW
    mesh = plsc.VectorSubcoreMesh(core_axis_name="c", subcore_axis_name="s",
                                  num_cores=2, num_subcores=16)
    return pl.kernel(
        _sc_body,
        out_type=(
            jax.ShapeDtypeStruct((B, K1), jnp.float32),
            jax.ShapeDtypeStruct((B, K1), jnp.float32),
            jax.ShapeDtypeStruct((B, D), jnp.float32),
            jax.ShapeDtypeStruct((B, D), jnp.float32),
        ),
        mesh=mesh,
        scratch_types=[
            pltpu.VMEM((bpw, n_chunks, CHUNK), jnp.int32),
            pltpu.VMEM((bpw, D), jnp.float32),
            pltpu.VMEM((bpw, D), jnp.float32),
            pltpu.VMEM((bpw,), jnp.int32),
            pltpu.VMEM((bpw, D), jnp.float32),
            pltpu.VMEM((CHUNK, D), jnp.float32),
            pltpu.VMEM((CHUNK, D), jnp.float32),
            pltpu.VMEM((CHUNK, D), jnp.float32),
            pltpu.VMEM((CHUNK, D), jnp.float32),
            pltpu.VMEM((CHUNK,), jnp.float32),
            pltpu.VMEM((CHUNK,), jnp.float32),
            pltpu.VMEM((CHUNK,), jnp.float32),
            pltpu.VMEM((CHUNK,), jnp.float32),
            pltpu.VMEM((LANES * LANES,), jnp.float32),
            pltpu.SemaphoreType.DMA,
            pltpu.SemaphoreType.DMA,
            pltpu.SemaphoreType.DMA,
            pltpu.SemaphoreType.DMA,
            pltpu.SemaphoreType.DMA,
            pltpu.SemaphoreType.DMA,
            pltpu.SemaphoreType.DMA,
            pltpu.SemaphoreType.DMA,
            pltpu.SemaphoreType.DMA,
        ],
        compiler_params=pltpu.CompilerParams(needs_layout_passes=False),
        interpret=interpret,
    )


LAG = 32


def _tc_update_body(y_ref, w_ref, g1_ref, g2_ref, v1_ref, v2_ref,
                    m1_ref, m2_ref, o1_ref, o2_ref,
                    u1_ref, u2_ref, sem1, sem2):
    del m1_ref, m2_ref
    B = y_ref.shape[0]
    t1 = g1_ref[...] * MOMENTUM + v1_ref[...] * (1.0 - MOMENTUM)
    n1 = jnp.sum(t1 * t1, axis=1, keepdims=True)
    u1_ref[...] = t1 / jnp.sqrt(n1)
    t2 = g2_ref[...] * MOMENTUM + v2_ref[...] * (1.0 - MOMENTUM)
    n2 = jnp.sum(t2 * t2, axis=1, keepdims=True)
    u2_ref[...] = t2 / jnp.sqrt(n2)

    def _wait_one():
        pltpu.make_async_copy(u1_ref.at[pl.ds(0, 1)], o1_ref.at[pl.ds(0, 1)], sem1).wait()
        pltpu.make_async_copy(u2_ref.at[pl.ds(0, 1)], o2_ref.at[pl.ds(0, 1)], sem2).wait()

    def sbody(i, carry):
        yi = y_ref[i]
        wi = w_ref[i]
        pltpu.make_async_copy(u1_ref.at[pl.ds(wi, 1)], o1_ref.at[pl.ds(yi, 1)], sem1).start()
        pltpu.make_async_copy(u2_ref.at[pl.ds(wi, 1)], o2_ref.at[pl.ds(yi, 1)], sem2).start()

        @pl.when(i >= LAG)
        def _():
            _wait_one()

        return carry

    lax.fori_loop(0, B, sbody, 0)

    def dbody(i, carry):
        _wait_one()
        return carry

    lax.fori_loop(0, min(LAG, B), dbody, 0)


def _make_tc_update(B, D, N, interpret=False):
    return pl.pallas_call(
        _tc_update_body,
        out_shape=(
            jax.ShapeDtypeStruct((N, D), jnp.float32),
            jax.ShapeDtypeStruct((N, D), jnp.float32),
        ),
        in_specs=[
            pl.BlockSpec(memory_space=pltpu.MemorySpace.SMEM),
            pl.BlockSpec(memory_space=pltpu.MemorySpace.SMEM),
            pl.BlockSpec(memory_space=pltpu.MemorySpace.VMEM),
            pl.BlockSpec(memory_space=pltpu.MemorySpace.VMEM),
            pl.BlockSpec(memory_space=pltpu.MemorySpace.VMEM),
            pl.BlockSpec(memory_space=pltpu.MemorySpace.VMEM),
            pl.BlockSpec(memory_space=pltpu.MemorySpace.HBM),
            pl.BlockSpec(memory_space=pltpu.MemorySpace.HBM),
        ],
        out_specs=(
            pl.BlockSpec(memory_space=pltpu.MemorySpace.HBM),
            pl.BlockSpec(memory_space=pltpu.MemorySpace.HBM),
        ),
        scratch_shapes=[
            pltpu.VMEM((B, D), jnp.float32),
            pltpu.VMEM((B, D), jnp.float32),
            pltpu.SemaphoreType.DMA,
            pltpu.SemaphoreType.DMA,
        ],
        input_output_aliases={6: 0, 7: 1},
        interpret=interpret,
    )


def _impl(v1, v2, y, idx, memory_v1, memory_v2, interpret=False):
    B, D = v1.shape
    K1 = idx.shape[1]
    N = memory_v1.shape[0]

    iota_b = jnp.arange(B, dtype=jnp.int32)
    w = jnp.zeros((N,), jnp.int32).at[y].max(iota_b)[y]

    sc_call = _make_sc_call(B, K1, D, N, interpret=interpret)
    out1, out2, g1, g2 = sc_call(
        v1.reshape(NW, B // NW, D), v2.reshape(NW, B // NW, D),
        idx.reshape(NW, B // NW, K1 // CHUNK, CHUNK),
        y.reshape(NW, B // NW),
        memory_v1, memory_v2,
    )

    tc_call = _make_tc_update(B, D, N, interpret=interpret)
    new1, new2 = tc_call(y, w, g1, g2, v1, v2, memory_v1, memory_v2)

    return (out1[:, :, None], out2[:, :, None], new1, new2)


def kernel(v1, v2, y, idx, memory_v1, memory_v2):
    return _impl(v1, v2, y, idx, memory_v1, memory_v2, interpret=False)


# speedup vs baseline: 10.5356x; 1.0550x over previous
import functools

import jax
import jax.numpy as jnp
from jax import lax
from jax.experimental import pallas as pl
from jax.experimental.pallas import tpu as pltpu
from jax.experimental.pallas import tpu_sc as plsc

NW = 32
CHUNK = 128
LANES = 16
T = 0.07
MOMENTUM = 0.5


def _iota16():
    return lax.iota(jnp.int32, LANES)


def _splat16(x):
    return jnp.full((LANES,), x, dtype=jnp.int32)


def _dot_chunk(buf, vv, out_v, c_base, scratch_v, col_idx, lane_base, n_groups):
    ones = jnp.ones((LANES,), dtype=jnp.int32)

    def g_body(g, carry):
        row0 = g * LANES
        for r in range(0, LANES, 4):
            rows = [row0 + r + j for j in range(4)]
            ld = [[buf[rw, pl.ds(p * LANES, LANES)] for p in range(8)]
                  for rw in rows]
            for j in range(4):
                a = [ld[j][p] * vv[p] for p in range(4)]
                for p in range(4, 8):
                    a[p - 4] = a[p - 4] + ld[j][p] * vv[p]
                scratch_v[pl.ds((r + j) * LANES, LANES)] = (a[0] + a[1]) + (a[2] + a[3])
        idxc = lane_base
        tot = plsc.load_gather(scratch_v, [idxc])
        for c in range(1, LANES):
            idxc = idxc + ones
            tot = tot + plsc.load_gather(scratch_v, [idxc])
        out_v[pl.ds(c_base + row0, LANES)] = tot
        return carry

    lax.fori_loop(0, n_groups, g_body, 0)


def _sc_body(v1_hbm, v2_hbm, idx_hbm, y_hbm,
             mem1_hbm, mem2_hbm,
             o1_hbm, o2_hbm, g1_hbm, g2_hbm,
             idx_all, vb1, vb2, y_v, gbuf_v,
             b1s0, b1s1, b2s0, b2s1,
             ob1s0, ob1s1, ob2s0, ob2s1, scratch_v,
             s1s0, s1s1, s2s0, s2s1,
             os1s0, os1s1, os2s0, os2s1, gsem):
    NWw, bpw, n_chunks, _ = idx_hbm.shape
    D = vb1.shape[1]
    wid = lax.axis_index("s") * 2 + lax.axis_index("c")
    inv_t = jnp.float32(1.0 / T)
    cshift = n_chunks.bit_length() - 1
    cmask = n_chunks - 1

    col_idx = [_iota16() + p * LANES for p in range(8)]
    lane_base = _iota16() * LANES
    n_groups = CHUNK // LANES

    pltpu.sync_copy(idx_hbm.at[wid], idx_all)
    pltpu.sync_copy(v1_hbm.at[wid], vb1)
    pltpu.sync_copy(v2_hbm.at[wid], vb2)

    pltpu.sync_copy(y_hbm.at[wid], y_v)
    pltpu.async_copy(mem1_hbm.at[y_v], gbuf_v, gsem).wait()
    pltpu.sync_copy(gbuf_v, g1_hbm.at[pl.ds(wid * bpw, bpw)])
    pltpu.async_copy(mem2_hbm.at[y_v], gbuf_v, gsem).wait()
    pltpu.sync_copy(gbuf_v, g2_hbm.at[pl.ds(wid * bpw, bpw)])

    bufs1 = (b1s0, b1s1)
    bufs2 = (b2s0, b2s1)
    sems1 = (s1s0, s1s1)
    sems2 = (s2s0, s2s1)
    obufs1 = (ob1s0, ob1s1)
    obufs2 = (ob2s0, ob2s1)
    osems1 = (os1s0, os1s1)
    osems2 = (os2s0, os2s1)

    n_units = bpw * n_chunks

    def unit_bc(u):
        return lax.shift_right_logical(u, cshift), jnp.bitwise_and(u, cmask)

    def start(u, slot):
        bl, c = unit_bc(u)
        pltpu.async_copy(mem1_hbm.at[idx_all.at[bl, c]], bufs1[slot], sems1[slot])
        pltpu.async_copy(mem2_hbm.at[idx_all.at[bl, c]], bufs2[slot], sems2[slot])

    def wait(u, slot):
        bl, c = unit_bc(u)
        pltpu.make_async_copy(mem1_hbm.at[idx_all.at[bl, c]], bufs1[slot], sems1[slot]).wait()
        pltpu.make_async_copy(mem2_hbm.at[idx_all.at[bl, c]], bufs2[slot], sems2[slot]).wait()

    def owait(par):
        pltpu.make_async_copy(obufs1[par], o1_hbm.at[0, pl.ds(0, CHUNK)], osems1[par]).wait()
        pltpu.make_async_copy(obufs2[par], o2_hbm.at[0, pl.ds(0, CHUNK)], osems2[par]).wait()

    def u_pair(up, carry2):
        for par in range(2):
            u = 2 * up + par

            @pl.when(u + 1 < n_units)
            def _():
                start(u + 1, 1 - par)

            wait(u, par)

            @pl.when(u >= 2)
            def _():
                owait(par)

            bl, c = unit_bc(u)
            vv1 = [vb1[bl, pl.ds(p * LANES, LANES)] * inv_t for p in range(8)]
            vv2 = [vb2[bl, pl.ds(p * LANES, LANES)] * inv_t for p in range(8)]
            _dot_chunk(bufs1[par], vv2, obufs2[par], 0, scratch_v,
                       col_idx, lane_base, n_groups)
            _dot_chunk(bufs2[par], vv1, obufs1[par], 0, scratch_v,
                       col_idx, lane_base, n_groups)
            b = wid * bpw + bl
            pltpu.async_copy(obufs1[par], o1_hbm.at[b, pl.ds(c * CHUNK, CHUNK)], osems1[par])
            pltpu.async_copy(obufs2[par], o2_hbm.at[b, pl.ds(c * CHUNK, CHUNK)], osems2[par])
        return carry2

    start(0, 0)
    lax.fori_loop(0, n_units // 2, u_pair, 0)
    owait(0)
    owait(1)


def _make_sc_call(B, K1, D, N, interpret=False):
    n_chunks = K1 // CHUNK
    bpw = B // NW
    mesh = plsc.VectorSubcoreMesh(core_axis_name="c", subcore_axis_name="s",
                                  num_cores=2, num_subcores=16)
    return pl.kernel(
        _sc_body,
        out_type=(
            jax.ShapeDtypeStruct((B, K1), jnp.float32),
            jax.ShapeDtypeStruct((B, K1), jnp.float32),
            jax.ShapeDtypeStruct((B, D), jnp.float32),
            jax.ShapeDtypeStruct((B, D), jnp.float32),
        ),
        mesh=mesh,
        scratch_types=[
            pltpu.VMEM((bpw, n_chunks, CHUNK), jnp.int32),
            pltpu.VMEM((bpw, D), jnp.float32),
            pltpu.VMEM((bpw, D), jnp.float32),
            pltpu.VMEM((bpw,), jnp.int32),
            pltpu.VMEM((bpw, D), jnp.float32),
            pltpu.VMEM((CHUNK, D), jnp.float32),
            pltpu.VMEM((CHUNK, D), jnp.float32),
            pltpu.VMEM((CHUNK, D), jnp.float32),
            pltpu.VMEM((CHUNK, D), jnp.float32),
            pltpu.VMEM((CHUNK,), jnp.float32),
            pltpu.VMEM((CHUNK,), jnp.float32),
            pltpu.VMEM((CHUNK,), jnp.float32),
            pltpu.VMEM((CHUNK,), jnp.float32),
            pltpu.VMEM((LANES * LANES,), jnp.float32),
            pltpu.SemaphoreType.DMA,
            pltpu.SemaphoreType.DMA,
            pltpu.SemaphoreType.DMA,
            pltpu.SemaphoreType.DMA,
            pltpu.SemaphoreType.DMA,
            pltpu.SemaphoreType.DMA,
            pltpu.SemaphoreType.DMA,
            pltpu.SemaphoreType.DMA,
            pltpu.SemaphoreType.DMA,
        ],
        compiler_params=pltpu.CompilerParams(needs_layout_passes=False),
        interpret=interpret,
    )


LAG = 32


def _tc_update_body(y_ref, w_ref, g1_ref, g2_ref, v1_ref, v2_ref,
                    m1_ref, m2_ref, o1_ref, o2_ref,
                    u1_ref, u2_ref, sem1, sem2):
    del m1_ref, m2_ref
    B = y_ref.shape[0]
    t1 = g1_ref[...] * MOMENTUM + v1_ref[...] * (1.0 - MOMENTUM)
    n1 = jnp.sum(t1 * t1, axis=1, keepdims=True)
    u1_ref[...] = t1 / jnp.sqrt(n1)
    t2 = g2_ref[...] * MOMENTUM + v2_ref[...] * (1.0 - MOMENTUM)
    n2 = jnp.sum(t2 * t2, axis=1, keepdims=True)
    u2_ref[...] = t2 / jnp.sqrt(n2)

    def _wait_one():
        pltpu.make_async_copy(u1_ref.at[pl.ds(0, 1)], o1_ref.at[pl.ds(0, 1)], sem1).wait()
        pltpu.make_async_copy(u2_ref.at[pl.ds(0, 1)], o2_ref.at[pl.ds(0, 1)], sem2).wait()

    def sbody(i, carry):
        yi = y_ref[i]
        wi = w_ref[i]
        pltpu.make_async_copy(u1_ref.at[pl.ds(wi, 1)], o1_ref.at[pl.ds(yi, 1)], sem1).start()
        pltpu.make_async_copy(u2_ref.at[pl.ds(wi, 1)], o2_ref.at[pl.ds(yi, 1)], sem2).start()

        @pl.when(i >= LAG)
        def _():
            _wait_one()

        return carry

    lax.fori_loop(0, B, sbody, 0)

    def dbody(i, carry):
        _wait_one()
        return carry

    lax.fori_loop(0, min(LAG, B), dbody, 0)


def _make_tc_update(B, D, N, interpret=False):
    return pl.pallas_call(
        _tc_update_body,
        out_shape=(
            jax.ShapeDtypeStruct((N, D), jnp.float32),
            jax.ShapeDtypeStruct((N, D), jnp.float32),
        ),
        in_specs=[
            pl.BlockSpec(memory_space=pltpu.MemorySpace.SMEM),
            pl.BlockSpec(memory_space=pltpu.MemorySpace.SMEM),
            pl.BlockSpec(memory_space=pltpu.MemorySpace.VMEM),
            pl.BlockSpec(memory_space=pltpu.MemorySpace.VMEM),
            pl.BlockSpec(memory_space=pltpu.MemorySpace.VMEM),
            pl.BlockSpec(memory_space=pltpu.MemorySpace.VMEM),
            pl.BlockSpec(memory_space=pltpu.MemorySpace.HBM),
            pl.BlockSpec(memory_space=pltpu.MemorySpace.HBM),
        ],
        out_specs=(
            pl.BlockSpec(memory_space=pltpu.MemorySpace.HBM),
            pl.BlockSpec(memory_space=pltpu.MemorySpace.HBM),
        ),
        scratch_shapes=[
            pltpu.VMEM((B, D), jnp.float32),
            pltpu.VMEM((B, D), jnp.float32),
            pltpu.SemaphoreType.DMA,
            pltpu.SemaphoreType.DMA,
        ],
        input_output_aliases={6: 0, 7: 1},
        interpret=interpret,
    )


def _impl(v1, v2, y, idx, memory_v1, memory_v2, interpret=False):
    B, D = v1.shape
    K1 = idx.shape[1]
    N = memory_v1.shape[0]

    iota_b = jnp.arange(B, dtype=jnp.int32)
    eq = y[:, None] == y[None, :]
    w = jnp.max(jnp.where(eq, iota_b[None, :], 0), axis=1).astype(jnp.int32)

    sc_call = _make_sc_call(B, K1, D, N, interpret=interpret)
    out1, out2, g1, g2 = sc_call(
        v1.reshape(NW, B // NW, D), v2.reshape(NW, B // NW, D),
        idx.reshape(NW, B // NW, K1 // CHUNK, CHUNK),
        y.reshape(NW, B // NW),
        memory_v1, memory_v2,
    )

    tc_call = _make_tc_update(B, D, N, interpret=interpret)
    new1, new2 = tc_call(y, w, g1, g2, v1, v2, memory_v1, memory_v2)

    return (out1[:, :, None], out2[:, :, None], new1, new2)


def kernel(v1, v2, y, idx, memory_v1, memory_v2):
    return _impl(v1, v2, y, idx, memory_v1, memory_v2, interpret=False)
